# baseline (speedup 1.0000x reference)
import functools

import jax
import jax.numpy as jnp
from jax import lax
from jax.experimental import pallas as pl
from jax.experimental.pallas import tpu as pltpu
from jax.experimental.pallas import tpu_sc as plsc

_EMBED_DIM = 32
_LANES = 16


def _body(b_per_w,
          uid_hbm, iid_hbm, uemb_hbm, iemb_hbm, ubias_hbm, ibias_hbm,
          wb_hbm, out_hbm,
          uid_v, iid_v, urows_v, irows_v, ub_v, ib_v, wb_v, out_v, sem):
    wid = lax.axis_index("s") * 2 + lax.axis_index("c")
    base = wid * b_per_w

    pltpu.sync_copy(uid_hbm.at[pl.ds(base, b_per_w)], uid_v)
    pltpu.sync_copy(iid_hbm.at[pl.ds(base, b_per_w)], iid_v)
    pltpu.sync_copy(wb_hbm, wb_v)

    cp_u = pltpu.async_copy(uemb_hbm.at[uid_v], urows_v, sem)
    cp_i = pltpu.async_copy(iemb_hbm.at[iid_v], irows_v, sem)
    cp_ub = pltpu.async_copy(ubias_hbm.at[uid_v], ub_v, sem)
    cp_ib = pltpu.async_copy(ibias_hbm.at[iid_v], ib_v, sem)
    cp_u.wait()
    cp_i.wait()
    cp_ub.wait()
    cp_ib.wait()

    wvecs = [wb_v[pl.ds(16 * k, 16)] for k in range(5)]

    def wsc(d):
        return wvecs[d // 16][d % 16]

    b_const = wsc(2 * _EMBED_DIM)
    lanes0 = lax.iota(jnp.int32, _LANES)

    def blk(i, carry):
        rbase = i * _LANES
        rows = lanes0 + rbase
        acc = ub_v[pl.ds(rbase, _LANES)] + ib_v[pl.ds(rbase, _LANES)] + b_const
        for d in range(_EMBED_DIM):
            col = jnp.full((_LANES,), d, jnp.int32)
            acc = acc + plsc.load_gather(urows_v, [rows, col]) * wsc(d)
        for d in range(_EMBED_DIM):
            col = jnp.full((_LANES,), d, jnp.int32)
            acc = acc + plsc.load_gather(irows_v, [rows, col]) * wsc(_EMBED_DIM + d)
        out_v[pl.ds(rbase, _LANES)] = jnp.clip(acc, 0.5, 5.0)
        return carry

    lax.fori_loop(0, b_per_w // _LANES, blk, 0)

    pltpu.sync_copy(out_v, out_hbm.at[pl.ds(base, b_per_w)])


def kernel(user_ids, item_ids, user_emb, item_emb, user_bias_tab, item_bias_tab, W, b):
    batch = user_ids.shape[0]
    n_workers = 32
    b_per_w = batch // n_workers

    wb = jnp.zeros((80,), jnp.float32).at[:2 * _EMBED_DIM].set(
        W.reshape(-1)).at[2 * _EMBED_DIM].set(b[0])
    ubias = user_bias_tab.reshape(-1)
    ibias = item_bias_tab.reshape(-1)

    mesh = plsc.VectorSubcoreMesh(core_axis_name="c", subcore_axis_name="s")
    out = pl.kernel(
        functools.partial(_body, b_per_w),
        out_type=jax.ShapeDtypeStruct((batch,), jnp.float32),
        mesh=mesh,
        compiler_params=pltpu.CompilerParams(
            needs_layout_passes=False, use_tc_tiling_on_sc=False),
        scratch_types=[
            pltpu.VMEM((b_per_w,), jnp.int32),
            pltpu.VMEM((b_per_w,), jnp.int32),
            pltpu.VMEM((b_per_w, _EMBED_DIM), jnp.float32),
            pltpu.VMEM((b_per_w, _EMBED_DIM), jnp.float32),
            pltpu.VMEM((b_per_w,), jnp.float32),
            pltpu.VMEM((b_per_w,), jnp.float32),
            pltpu.VMEM((80,), jnp.float32),
            pltpu.VMEM((b_per_w,), jnp.float32),
            pltpu.SemaphoreType.DMA,
        ],
    )(user_ids, item_ids, user_emb, item_emb, ubias, ibias, wb)
    return out.reshape(batch, 1)


# speedup vs baseline: 8.2959x; 8.2959x over previous
import functools

import jax
import jax.numpy as jnp
from jax import lax
from jax.experimental import pallas as pl
from jax.experimental.pallas import tpu as pltpu
from jax.experimental.pallas import tpu_sc as plsc

_D = 32
_LANES = 16
_BC = 32768


def _phase1(ut_ref, it_ref, ub_ref, ib_ref, wu_ref, wi_ref, su_ref, si_ref):
    su_ref[...] = jnp.sum(ut_ref[...] * wu_ref[...], axis=0) + ub_ref[0, :]
    si_ref[...] = jnp.sum(it_ref[...] * wi_ref[...], axis=0) + ib_ref[0, :]


def _phase2(b_per_w,
            uid_hbm, iid_hbm, su_hbm, si_hbm, b16_hbm,
            out_hbm,
            uid_v, iid_v, sug_v, sig_v, b_v, out_v, sem):
    wid = lax.axis_index("s") * 2 + lax.axis_index("c")
    base = wid * b_per_w

    pltpu.sync_copy(uid_hbm.at[pl.ds(base, b_per_w)], uid_v)
    pltpu.sync_copy(iid_hbm.at[pl.ds(base, b_per_w)], iid_v)
    pltpu.sync_copy(b16_hbm, b_v)

    cp_u = pltpu.async_copy(su_hbm.at[uid_v], sug_v, sem)
    cp_i = pltpu.async_copy(si_hbm.at[iid_v], sig_v, sem)
    cp_u.wait()
    cp_i.wait()

    b_vec = b_v[...]

    def blk(i, carry):
        rbase = i * _LANES
        acc = sug_v[pl.ds(rbase, _LANES)] + sig_v[pl.ds(rbase, _LANES)]
        out_v[pl.ds(rbase, _LANES)] = jnp.clip(acc + b_vec, 0.5, 5.0)
        return carry

    lax.fori_loop(0, b_per_w // _LANES, blk, 0)

    pltpu.sync_copy(out_v, out_hbm.at[pl.ds(base, b_per_w)])


def kernel(user_ids, item_ids, user_emb, item_emb, user_bias_tab, item_bias_tab, W, b):
    batch = user_ids.shape[0]
    n_workers = 32
    b_per_w = batch // n_workers
    n_rows = user_emb.shape[0]

    ut = user_emb.T
    it = item_emb.T
    ub_t = user_bias_tab.T
    ib_t = item_bias_tab.T
    wu = W[0, :_D].reshape(_D, 1)
    wi = W[0, _D:].reshape(_D, 1)
    b16 = jnp.full((_LANES,), b[0], jnp.float32)

    nb = (n_rows + _BC - 1) // _BC
    su, si = pl.pallas_call(
        _phase1,
        grid=(nb,),
        in_specs=[
            pl.BlockSpec((_D, _BC), lambda i: (0, i)),
            pl.BlockSpec((_D, _BC), lambda i: (0, i)),
            pl.BlockSpec((1, _BC), lambda i: (0, i)),
            pl.BlockSpec((1, _BC), lambda i: (0, i)),
            pl.BlockSpec((_D, 1), lambda i: (0, 0)),
            pl.BlockSpec((_D, 1), lambda i: (0, 0)),
        ],
        out_specs=[
            pl.BlockSpec((_BC,), lambda i: (i,)),
            pl.BlockSpec((_BC,), lambda i: (i,)),
        ],
        out_shape=[
            jax.ShapeDtypeStruct((n_rows,), jnp.float32),
            jax.ShapeDtypeStruct((n_rows,), jnp.float32),
        ],
    )(ut, it, ub_t, ib_t, wu, wi)

    mesh = plsc.VectorSubcoreMesh(core_axis_name="c", subcore_axis_name="s")
    out = pl.kernel(
        functools.partial(_phase2, b_per_w),
        out_type=jax.ShapeDtypeStruct((batch,), jnp.float32),
        mesh=mesh,
        compiler_params=pltpu.CompilerParams(
            needs_layout_passes=False, use_tc_tiling_on_sc=False),
        scratch_types=[
            pltpu.VMEM((b_per_w,), jnp.int32),
            pltpu.VMEM((b_per_w,), jnp.int32),
            pltpu.VMEM((b_per_w,), jnp.float32),
            pltpu.VMEM((b_per_w,), jnp.float32),
            pltpu.VMEM((_LANES,), jnp.float32),
            pltpu.VMEM((b_per_w,), jnp.float32),
            pltpu.SemaphoreType.DMA,
        ],
    )(user_ids, item_ids, su, si, b16)
    return out.reshape(batch, 1)


# speedup vs baseline: 8.4076x; 1.0135x over previous
import functools

import jax
import jax.numpy as jnp
from jax import lax
from jax.experimental import pallas as pl
from jax.experimental.pallas import tpu as pltpu
from jax.experimental.pallas import tpu_sc as plsc

_D = 32
_LANES = 16
_BC = 49152


def _phase1(ut_ref, it_ref, ub_ref, ib_ref, wu_ref, wi_ref, su_ref, si_ref):
    su_ref[...] = jnp.sum(ut_ref[...] * wu_ref[...], axis=0) + ub_ref[0, :]
    si_ref[...] = jnp.sum(it_ref[...] * wi_ref[...], axis=0) + ib_ref[0, :]


def _phase2(b_per_w,
            uid_hbm, iid_hbm, su_hbm, si_hbm, b16_hbm,
            out_hbm,
            uid_v, iid_v, sug_v, sig_v, b_v, out_v, sem):
    wid = lax.axis_index("s") * 2 + lax.axis_index("c")
    base = wid * b_per_w

    pltpu.sync_copy(uid_hbm.at[pl.ds(base, b_per_w)], uid_v)
    pltpu.sync_copy(iid_hbm.at[pl.ds(base, b_per_w)], iid_v)
    pltpu.sync_copy(b16_hbm, b_v)

    cp_u = pltpu.async_copy(su_hbm.at[uid_v], sug_v, sem)
    cp_i = pltpu.async_copy(si_hbm.at[iid_v], sig_v, sem)
    cp_u.wait()
    cp_i.wait()

    b_vec = b_v[...]

    def blk(i, carry):
        rbase = i * _LANES
        acc = sug_v[pl.ds(rbase, _LANES)] + sig_v[pl.ds(rbase, _LANES)]
        out_v[pl.ds(rbase, _LANES)] = jnp.clip(acc + b_vec, 0.5, 5.0)
        return carry

    lax.fori_loop(0, b_per_w // _LANES, blk, 0)

    pltpu.sync_copy(out_v, out_hbm.at[pl.ds(base, b_per_w)])


def kernel(user_ids, item_ids, user_emb, item_emb, user_bias_tab, item_bias_tab, W, b):
    batch = user_ids.shape[0]
    n_workers = 32
    b_per_w = batch // n_workers
    n_rows = user_emb.shape[0]

    ut = user_emb.T
    it = item_emb.T
    ub_t = user_bias_tab.T
    ib_t = item_bias_tab.T
    wu = W[0, :_D].reshape(_D, 1)
    wi = W[0, _D:].reshape(_D, 1)
    b16 = jnp.full((_LANES,), b[0], jnp.float32)

    nb = (n_rows + _BC - 1) // _BC
    su, si = pl.pallas_call(
        _phase1,
        grid=(nb,),
        in_specs=[
            pl.BlockSpec((_D, _BC), lambda i: (0, i)),
            pl.BlockSpec((_D, _BC), lambda i: (0, i)),
            pl.BlockSpec((1, _BC), lambda i: (0, i)),
            pl.BlockSpec((1, _BC), lambda i: (0, i)),
            pl.BlockSpec((_D, 1), lambda i: (0, 0)),
            pl.BlockSpec((_D, 1), lambda i: (0, 0)),
        ],
        out_specs=[
            pl.BlockSpec((_BC,), lambda i: (i,)),
            pl.BlockSpec((_BC,), lambda i: (i,)),
        ],
        out_shape=[
            jax.ShapeDtypeStruct((n_rows,), jnp.float32),
            jax.ShapeDtypeStruct((n_rows,), jnp.float32),
        ],
    )(ut, it, ub_t, ib_t, wu, wi)

    mesh = plsc.VectorSubcoreMesh(core_axis_name="c", subcore_axis_name="s")
    out = pl.kernel(
        functools.partial(_phase2, b_per_w),
        out_type=jax.ShapeDtypeStruct((batch,), jnp.float32),
        mesh=mesh,
        compiler_params=pltpu.CompilerParams(
            needs_layout_passes=False, use_tc_tiling_on_sc=False),
        scratch_types=[
            pltpu.VMEM((b_per_w,), jnp.int32),
            pltpu.VMEM((b_per_w,), jnp.int32),
            pltpu.VMEM((b_per_w,), jnp.float32),
            pltpu.VMEM((b_per_w,), jnp.float32),
            pltpu.VMEM((_LANES,), jnp.float32),
            pltpu.VMEM((b_per_w,), jnp.float32),
            pltpu.SemaphoreType.DMA,
        ],
    )(user_ids, item_ids, su, si, b16)
    return out.reshape(batch, 1)
